# baseline (speedup 1.0000x reference)
import jax
import jax.numpy as jnp
from jax.experimental import pallas as pl

_ROWS = 1000


def _mlp3_kernel(x_ref, w0_ref, b0_ref, w1_ref, b1_ref, w2_ref, b2_ref, o_ref):
    h = jnp.dot(x_ref[...], w0_ref[...], preferred_element_type=jnp.float32)
    h = jnp.maximum(h + b0_ref[...], 0.0)
    h = jnp.dot(h, w1_ref[...], preferred_element_type=jnp.float32)
    h = jnp.maximum(h + b1_ref[...], 0.0)
    h = jnp.dot(h, w2_ref[...], preferred_element_type=jnp.float32)
    o_ref[...] = h + b2_ref[...]


def kernel(x, weight, W0, b0, W1, b1, W2, b2, edge_index, batch):
    n, d_in = x.shape
    hid = W0.shape[1]
    d_out = W2.shape[1]
    grid = (n // _ROWS,)
    full = lambda shape: pl.BlockSpec(shape, lambda i: (0,) * len(shape))
    return pl.pallas_call(
        _mlp3_kernel,
        grid=grid,
        in_specs=[
            pl.BlockSpec((_ROWS, d_in), lambda i: (i, 0)),
            full((d_in, hid)),
            full((1, hid)),
            full((hid, hid)),
            full((1, hid)),
            full((hid, d_out)),
            full((1, d_out)),
        ],
        out_specs=pl.BlockSpec((_ROWS, d_out), lambda i: (i, 0)),
        out_shape=jax.ShapeDtypeStruct((n, d_out), x.dtype),
    )(x, W0, b0.reshape(1, hid), W1, b1.reshape(1, hid), W2, b2.reshape(1, d_out))
